# Initial kernel scaffold; baseline (speedup 1.0000x reference)
import jax
import jax.numpy as jnp
from jax.experimental import pallas as pl


def kernel(x, edge_index, reranker_scores, W_l, b_l, W_r, w_score, b_score, alpha):
    raise NotImplementedError("write your pallas kernel here")


# speedup vs baseline: 9.6278x; 9.6278x over previous
import functools

import jax
import jax.numpy as jnp
from jax import lax
from jax.experimental import pallas as pl
from jax.experimental.pallas import tpu as pltpu
from jax.experimental.pallas import tpu_sc as plsc

_N = 10000
_E = 320000
_D = 128
_CW = 16
_NW = 32
_EPW = _E // _NW
_K = 125
_NCH = _EPW // _K
_NBUF = 4
_NP = 10240
_RPT = _NP // 16


def _seg_body(x_hbm, src_hbm, dst_hbm, zf_hbm, zc_hbm, ones_hbm,
              pf_hbm, pc_hbm,
              acc, cacc, dst_v, ones_v, i0, i1, b0, b1,
              si0, si1, s0, s1):
    cid = lax.axis_index("c")
    sid = lax.axis_index("s")
    wid = sid * 2 + cid
    ibufs = (i0, i1)
    bufs = (b0, b1)
    isems = (si0, si1)
    sems = (s0, s1)

    pltpu.sync_copy(dst_hbm.at[wid], dst_v)
    pltpu.sync_copy(ones_hbm, ones_v)

    base = sid * _RPT
    pltpu.sync_copy(zf_hbm, acc.at[pl.ds(base, _RPT)])
    pltpu.sync_copy(zc_hbm, cacc.at[pl.ds(base, _RPT)])
    plsc.subcore_barrier()

    def group(g, carry):
        c0 = g * 2
        hi = [pltpu.async_copy(src_hbm.at[wid].at[c0 + b], ibufs[b], isems[b])
              for b in range(2)]
        hr = []
        for b in range(2):
            hi[b].wait()
            hr.append(pltpu.async_copy(x_hbm.at[ibufs[b]], bufs[b], sems[b]))
        for b in range(2):
            hr[b].wait()
            pltpu.sync_copy(bufs[b], acc.at[dst_v.at[c0 + b]], add=True)
            pltpu.sync_copy(ones_v, cacc.at[dst_v.at[c0 + b]], add=True)
        return carry

    lax.fori_loop(0, _NCH // 2, group, 0)
    plsc.subcore_barrier()

    pltpu.sync_copy(acc.at[pl.ds(base, _RPT)],
                    pf_hbm.at[cid].at[pl.ds(base, _RPT)])
    pltpu.sync_copy(cacc.at[pl.ds(base, _RPT)],
                    pc_hbm.at[cid].at[pl.ds(base, _RPT)])


@functools.cache
def _make_seg():
  return pl.kernel(
    _seg_body,
    out_type=(jax.ShapeDtypeStruct((2, _NP, _D), jnp.float32),
              jax.ShapeDtypeStruct((2, _NP), jnp.float32)),
    mesh=plsc.VectorSubcoreMesh(core_axis_name="c", subcore_axis_name="s"),
    scratch_types=[
        pltpu.VMEM_SHARED((_NP, _D), jnp.float32),
        pltpu.VMEM_SHARED((_NP,), jnp.float32),
        pltpu.VMEM((_NCH, _K), jnp.int32),
        pltpu.VMEM((_K,), jnp.float32),
        pltpu.VMEM((_K,), jnp.int32),
        pltpu.VMEM((_K,), jnp.int32),
        pltpu.VMEM((_K, _D), jnp.float32),
        pltpu.VMEM((_K, _D), jnp.float32),
        pltpu.SemaphoreType.DMA,
        pltpu.SemaphoreType.DMA,
        pltpu.SemaphoreType.DMA,
        pltpu.SemaphoreType.DMA,
    ],
  )


def _post_body(pf_ref, cnt_ref, x_ref, wl_ref, bl_ref, wr_ref, ws_ref,
               bs_ref, al_ref, rs_ref, out_ref):
    seg = pf_ref[0] + pf_ref[1]
    mean = seg / jnp.maximum(cnt_ref[...], 1.0)
    x = x_ref[...]
    pre = (jnp.dot(mean, wl_ref[...], preferred_element_type=jnp.float32)
           + bl_ref[...]
           + jnp.dot(x, wr_ref[...], preferred_element_type=jnp.float32))
    h = jnp.maximum(pre, 0.0) + x
    sc = jnp.dot(h, ws_ref[...], preferred_element_type=jnp.float32) + bs_ref[...]
    a = jax.nn.sigmoid(al_ref[...])
    out_ref[...] = a * rs_ref[...] + (1.0 - a) * sc


_post = pl.pallas_call(
    _post_body,
    out_shape=jax.ShapeDtypeStruct((_N, 1), jnp.float32),
    grid=(1,),
    in_specs=[
        pl.BlockSpec((2, _N, _D), lambda i: (0, 0, 0)),
        pl.BlockSpec((_N, 1), lambda i: (0, 0)),
        pl.BlockSpec((_N, _D), lambda i: (0, 0)),
        pl.BlockSpec((_D, _D), lambda i: (0, 0)),
        pl.BlockSpec((1, _D), lambda i: (0, 0)),
        pl.BlockSpec((_D, _D), lambda i: (0, 0)),
        pl.BlockSpec((_D, 1), lambda i: (0, 0)),
        pl.BlockSpec((1, 1), lambda i: (0, 0)),
        pl.BlockSpec((1, 1), lambda i: (0, 0)),
        pl.BlockSpec((_N, 1), lambda i: (0, 0)),
    ],
    out_specs=pl.BlockSpec((_N, 1), lambda i: (0, 0)),
)


@jax.jit
def kernel(x, edge_index, reranker_scores, W_l, b_l, W_r, w_score, b_score,
           alpha):
    src = edge_index[0].reshape(_NW, _NCH, _K)
    dst = edge_index[1].reshape(_NW, _NCH, _K)
    zf = jnp.zeros((_RPT, _D), jnp.float32)
    zc = jnp.zeros((_RPT,), jnp.float32)
    ones1 = jnp.ones((_K,), jnp.float32)
    pf, pcnt = _make_seg()(x, src, dst, zf, zc, ones1)
    cnt = (pcnt[0, :_N] + pcnt[1, :_N]).reshape(_N, 1)
    out = _post(pf, cnt, x, W_l, b_l.reshape(1, _D), W_r, w_score,
                b_score.reshape(1, 1), alpha.reshape(1, 1),
                reranker_scores.reshape(_N, 1))
    return out[:, 0]
